# baseline (speedup 1.0000x reference)
import functools

import jax
import jax.numpy as jnp
from jax import lax
from jax.experimental import pallas as pl
from jax.experimental.pallas import tpu as pltpu
from jax.experimental.pallas import tpu_sc as plsc

BLK = 256
BLK_SHIFT = 8
SENTINEL = 1 << 28


def _router_kernel(x_ref, wg_ref, slots_ref, wrows_ref, meta_ref, *,
                   n_experts, n_blocks, seq):
    E = n_experts
    logits = lax.dot_general(x_ref[...], wg_ref[...],
                             (((1,), (1,)), ((), ())),
                             preferred_element_type=jnp.float32)
    m = jnp.max(logits, axis=-1, keepdims=True)
    p = jnp.exp(logits - m)
    p = p / jnp.sum(p, axis=-1, keepdims=True)
    lane = lax.broadcasted_iota(jnp.int32, p.shape, 1)
    m1 = jnp.max(p, axis=-1, keepdims=True)
    i1 = jnp.min(jnp.where(p == m1, lane, E), axis=-1, keepdims=True)
    mask1 = lane == i1
    p2 = jnp.where(mask1, -jnp.inf, p)
    m2 = jnp.max(p2, axis=-1, keepdims=True)
    i2 = jnp.min(jnp.where(p2 == m2, lane, E), axis=-1, keepdims=True)
    mask2 = lane == i2
    denom = m1 + m2
    w1n = m1 / denom
    w2n = m2 / denom

    cnt = (jnp.where(mask1, 1.0, 0.0) + jnp.where(mask2, 1.0, 0.0))
    csum = cnt
    shift = 1
    while shift < seq:
        shifted = jnp.concatenate(
            [jnp.zeros((shift, E), jnp.float32), csum[:-shift, :]], axis=0)
        csum = csum + shifted
        shift *= 2

    counts = csum[seq - 1:seq, :].astype(jnp.int32)
    pb = (counts + BLK - 1) >> BLK_SHIFT

    lane1 = lax.broadcasted_iota(jnp.int32, (1, E), 1)
    off_blk = jnp.zeros((1, E), jnp.int32)
    acc = jnp.zeros((), jnp.int32)
    for e in range(E):
        off_blk = jnp.where(lane1 == e, acc, off_blk)
        acc = acc + pb[0, e]
    nact = acc

    off_rows = (off_blk << BLK_SHIFT).astype(jnp.float32)
    slot_base = off_rows + csum - 1.0
    slot0 = jnp.sum(jnp.where(mask1, slot_base, 0.0), axis=-1,
                    keepdims=True).astype(jnp.int32)
    slot1 = jnp.sum(jnp.where(mask2, slot_base, 0.0), axis=-1,
                    keepdims=True).astype(jnp.int32)

    slots_ref[...] = jnp.concatenate(
        [lax.transpose(slot0, (1, 0)), lax.transpose(slot1, (1, 0))], axis=0)
    wrows_ref[...] = jnp.concatenate(
        [jnp.broadcast_to(w1n, (seq, 128)),
         jnp.broadcast_to(w2n, (seq, 128))], axis=0)

    JMAX = seq // BLK
    lane64 = lax.broadcasted_iota(jnp.int32, (1, E * JMAX), 1)
    e_l = lane64 // JMAX
    j_l = lane64 % JMAX
    off_l = jnp.zeros((1, E * JMAX), jnp.int32)
    pb_l = jnp.zeros((1, E * JMAX), jnp.int32)
    for e in range(E):
        off_l = jnp.where(e_l == e, off_blk[0, e], off_l)
        pb_l = jnp.where(e_l == e, pb[0, e], pb_l)
    xgi = jnp.minimum(off_l + jnp.minimum(j_l, jnp.maximum(pb_l - 1, 0)),
                      n_blocks - 1)
    lane_m = lax.broadcasted_iota(jnp.int32, (1, E * JMAX + E), 1)
    pb_tail = jnp.zeros((1, E * JMAX + E), jnp.int32)
    for e in range(E):
        pb_tail = jnp.where(lane_m == E * JMAX + e, pb[0, e], pb_tail)
    meta_ref[...] = jnp.where(lane_m < E * JMAX,
                              jnp.pad(xgi, ((0, 0), (0, E))), pb_tail)


def _gemm_kernel(meta_ref, wvec_ref, xg_ref, w1_ref, b1_ref,
                 w2_ref, b2_ref, ygw_ref, *, n_blocks, seq):
    e = pl.program_id(0)
    j = pl.program_id(1)
    jmax = seq // BLK
    pb_e = meta_ref[8 * jmax + e]

    @pl.when(j < pb_e)
    def _active():
        wrow = wvec_ref[:, 0:1]
        h = lax.dot_general(xg_ref[...].astype(jnp.bfloat16), w1_ref[0],
                            (((1,), (1,)), ((), ())),
                            preferred_element_type=jnp.float32)
        h = h + b1_ref[0]
        h = h * jax.nn.sigmoid(h)
        y = lax.dot_general(h.astype(jnp.bfloat16), w2_ref[0],
                            (((1,), (1,)), ((), ())),
                            preferred_element_type=jnp.float32)
        ygw_ref[...] = (y + b2_ref[0]) * wrow


def _add_kernel(y0_ref, y1_ref, out_ref):
    out_ref[...] = (y0_ref[...].astype(jnp.float32) +
                    y1_ref[...].astype(jnp.float32))


def kernel(x, Wg, W1, b1, W2, b2):
    B_, S, H = x.shape
    E, F, _ = W1.shape
    NB = (2 * S + E * (BLK - 1)) // BLK
    G = NB * BLK

    xs = x[0]
    w1 = W1.astype(jnp.bfloat16)
    w2 = W2.astype(jnp.bfloat16)

    slots2, wrows, meta2d = pl.pallas_call(
        functools.partial(_router_kernel, n_experts=E, n_blocks=NB, seq=S),
        in_specs=[pl.BlockSpec((S, H), lambda: (0, 0)),
                  pl.BlockSpec((E, H), lambda: (0, 0))],
        out_specs=[pl.BlockSpec((2, S), lambda: (0, 0)),
                   pl.BlockSpec((2 * S, 128), lambda: (0, 0)),
                   pl.BlockSpec((1, E * (S // BLK) + E), lambda: (0, 0))],
        out_shape=[jax.ShapeDtypeStruct((2, S), jnp.int32),
                   jax.ShapeDtypeStruct((2 * S, 128), jnp.float32),
                   jax.ShapeDtypeStruct((1, E * (S // BLK) + E), jnp.int32)],
    )(xs, Wg)

    JMAX = S // BLK
    meta = meta2d.reshape(E * JMAX + E)

    mesh = plsc.VectorSubcoreMesh(core_axis_name="c", subcore_axis_name="s")
    NW = 32
    tok_per = S // NW

    @functools.partial(
        pl.kernel, mesh=mesh,
        out_type=[jax.ShapeDtypeStruct((G, H), jnp.float32),
                  jax.ShapeDtypeStruct((G, 128), jnp.float32)],
        scratch_types=[pltpu.VMEM((tok_per,), jnp.int32),
                       pltpu.VMEM((tok_per,), jnp.int32),
                       pltpu.VMEM((tok_per, H), jnp.float32),
                       pltpu.VMEM((tok_per, 128), jnp.float32),
                       pltpu.VMEM((tok_per, 128), jnp.float32),
                       pltpu.SemaphoreType.DMA,
                       pltpu.SemaphoreType.DMA])
    def _dispatch(x_hbm, s_hbm, w_hbm, xg_hbm, wv_hbm,
                  idx0_v, idx1_v, rows_v, w0_v, w1_v, lsem, ssem):
        wid = lax.axis_index("s") * 2 + lax.axis_index("c")
        base = wid * tok_per
        l0 = pltpu.async_copy(s_hbm.at[0, pl.ds(base, tok_per)], idx0_v, lsem)
        l1 = pltpu.async_copy(s_hbm.at[1, pl.ds(base, tok_per)], idx1_v, lsem)
        l2 = pltpu.async_copy(x_hbm.at[pl.ds(base, tok_per)], rows_v, lsem)
        l3 = pltpu.async_copy(w_hbm.at[pl.ds(base, tok_per)], w0_v, lsem)
        l4 = pltpu.async_copy(w_hbm.at[pl.ds(S + base, tok_per)], w1_v, lsem)
        l0.wait(); l1.wait(); l2.wait(); l3.wait(); l4.wait()
        s0 = pltpu.async_copy(rows_v, xg_hbm.at[idx0_v], ssem)
        s1 = pltpu.async_copy(rows_v, xg_hbm.at[idx1_v], ssem)
        s2 = pltpu.async_copy(w0_v, wv_hbm.at[idx0_v], ssem)
        s3 = pltpu.async_copy(w1_v, wv_hbm.at[idx1_v], ssem)
        s0.wait(); s1.wait(); s2.wait(); s3.wait()

    xg, wvec = _dispatch(xs, slots2, wrows)

    ygw = pl.pallas_call(
        functools.partial(_gemm_kernel, n_blocks=NB, seq=S),
        grid_spec=pltpu.PrefetchScalarGridSpec(
            num_scalar_prefetch=1,
            grid=(E, JMAX),
            in_specs=[
                pl.BlockSpec((BLK, 128),
                             lambda e, j, meta: (meta[e * (S // BLK) + j], 0)),
                pl.BlockSpec((BLK, H),
                             lambda e, j, meta: (meta[e * (S // BLK) + j], 0)),
                pl.BlockSpec((1, F, H), lambda e, j, meta: (e, 0, 0)),
                pl.BlockSpec((1, 1, F), lambda e, j, meta: (e, 0, 0)),
                pl.BlockSpec((1, H, F), lambda e, j, meta: (e, 0, 0)),
                pl.BlockSpec((1, 1, H), lambda e, j, meta: (e, 0, 0)),
            ],
            out_specs=pl.BlockSpec(
                (BLK, H), lambda e, j, meta: (meta[e * (S // BLK) + j], 0)),
        ),
        compiler_params=pltpu.CompilerParams(
            dimension_semantics=("arbitrary", "arbitrary")),
        out_shape=jax.ShapeDtypeStruct((G, H), jnp.float32),
    )(meta, wvec, xg, w1, b1.reshape(E, 1, F), w2,
      b2.reshape(E, 1, H))

    CH = 64

    @functools.partial(
        pl.kernel, mesh=mesh,
        out_type=jax.ShapeDtypeStruct((2 * S, H), jnp.float32),
        scratch_types=[pltpu.VMEM((CH,), jnp.int32),
                       pltpu.VMEM((CH, H), jnp.float32)])
    def _combine(ygw_hbm, s_hbm, y_hbm, idx_v, rows_v):
        wid = lax.axis_index("s") * 2 + lax.axis_index("c")
        k = wid % 2
        tchunk = wid // 2
        for c in range(2):
            base = tchunk * 2 * CH + c * CH
            pltpu.sync_copy(s_hbm.at[k, pl.ds(base, CH)], idx_v)
            pltpu.sync_copy(ygw_hbm.at[idx_v], rows_v)
            pltpu.sync_copy(rows_v, y_hbm.at[pl.ds(k * S + base, CH)])

    y01 = _combine(ygw, slots2)

    TS = 1024
    NT = S // TS
    out = pl.pallas_call(
        _add_kernel,
        grid=(NT,),
        in_specs=[pl.BlockSpec((TS, H), lambda t: (t, 0)),
                  pl.BlockSpec((TS, H), lambda t: (t + NT, 0))],
        out_specs=pl.BlockSpec((TS, H), lambda t: (t, 0)),
        compiler_params=pltpu.CompilerParams(
            dimension_semantics=("parallel",)),
        out_shape=jax.ShapeDtypeStruct((S, H), jnp.float32),
    )(y01, y01)
    return out[None]


# speedup vs baseline: 1.1322x; 1.1322x over previous
import functools

import jax
import jax.numpy as jnp
from jax import lax
from jax.experimental import pallas as pl
from jax.experimental.pallas import tpu as pltpu
from jax.experimental.pallas import tpu_sc as plsc

BLK = 512
BLK_SHIFT = 9
SENTINEL = 1 << 28


def _router_kernel(x_ref, wg_ref, slots_ref, wrows_ref, meta_ref, *,
                   n_experts, n_blocks, seq):
    E = n_experts
    logits = lax.dot_general(x_ref[...], wg_ref[...],
                             (((1,), (1,)), ((), ())),
                             preferred_element_type=jnp.float32)
    m = jnp.max(logits, axis=-1, keepdims=True)
    p = jnp.exp(logits - m)
    p = p / jnp.sum(p, axis=-1, keepdims=True)
    lane = lax.broadcasted_iota(jnp.int32, p.shape, 1)
    m1 = jnp.max(p, axis=-1, keepdims=True)
    i1 = jnp.min(jnp.where(p == m1, lane, E), axis=-1, keepdims=True)
    mask1 = lane == i1
    p2 = jnp.where(mask1, -jnp.inf, p)
    m2 = jnp.max(p2, axis=-1, keepdims=True)
    i2 = jnp.min(jnp.where(p2 == m2, lane, E), axis=-1, keepdims=True)
    mask2 = lane == i2
    denom = m1 + m2
    w1n = m1 / denom
    w2n = m2 / denom

    cnt = (jnp.where(mask1, 1.0, 0.0) + jnp.where(mask2, 1.0, 0.0))
    csum = cnt
    shift = 1
    while shift < seq:
        shifted = jnp.concatenate(
            [jnp.zeros((shift, E), jnp.float32), csum[:-shift, :]], axis=0)
        csum = csum + shifted
        shift *= 2

    counts = csum[seq - 1:seq, :].astype(jnp.int32)
    pb = (counts + BLK - 1) >> BLK_SHIFT

    lane1 = lax.broadcasted_iota(jnp.int32, (1, E), 1)
    off_blk = jnp.zeros((1, E), jnp.int32)
    acc = jnp.zeros((), jnp.int32)
    for e in range(E):
        off_blk = jnp.where(lane1 == e, acc, off_blk)
        acc = acc + pb[0, e]
    nact = acc

    off_rows = (off_blk << BLK_SHIFT).astype(jnp.float32)
    slot_base = off_rows + csum - 1.0
    slot0 = jnp.sum(jnp.where(mask1, slot_base, 0.0), axis=-1,
                    keepdims=True).astype(jnp.int32)
    slot1 = jnp.sum(jnp.where(mask2, slot_base, 0.0), axis=-1,
                    keepdims=True).astype(jnp.int32)

    slots_ref[...] = jnp.concatenate(
        [lax.transpose(slot0, (1, 0)), lax.transpose(slot1, (1, 0))], axis=0)
    wrows_ref[...] = jnp.concatenate(
        [jnp.broadcast_to(w1n, (seq, 128)),
         jnp.broadcast_to(w2n, (seq, 128))], axis=0)

    lane_nb = lax.broadcasted_iota(jnp.int32, (1, n_blocks + 1), 1)
    ii = jnp.minimum(lane_nb, nact - 1)
    eob = jnp.zeros((1, n_blocks + 1), jnp.int32)
    for e in range(E):
        eob = eob + jnp.where(off_blk[0, e] <= ii, 1, 0)
    eob = eob - 1
    meta_ref[...] = jnp.where(lane_nb == n_blocks, nact, eob)


def _gemm_kernel(meta_ref, wvec_ref, xg_ref, w1_ref, b1_ref,
                 w2_ref, b2_ref, ygw_ref, *, n_blocks, seq):
    i = pl.program_id(0)
    nact = meta_ref[n_blocks]

    @pl.when(i < nact)
    def _active():
        wrow = wvec_ref[:, 0:1]
        h = lax.dot_general(xg_ref[...].astype(jnp.bfloat16), w1_ref[0],
                            (((1,), (1,)), ((), ())),
                            preferred_element_type=jnp.float32)
        h = h + b1_ref[0]
        h = h * jax.nn.sigmoid(h)
        y = lax.dot_general(h.astype(jnp.bfloat16), w2_ref[0],
                            (((1,), (1,)), ((), ())),
                            preferred_element_type=jnp.float32)
        ygw_ref[...] = (y + b2_ref[0]) * wrow

    @pl.when(i >= nact)
    def _inactive():
        ygw_ref[...] = jnp.zeros_like(ygw_ref)


def _add_kernel(y0_ref, y1_ref, out_ref):
    out_ref[...] = (y0_ref[...].astype(jnp.float32) +
                    y1_ref[...].astype(jnp.float32))


def kernel(x, Wg, W1, b1, W2, b2):
    B_, S, H = x.shape
    E, F, _ = W1.shape
    NB = (2 * S + E * (BLK - 1)) // BLK
    G = NB * BLK

    xs = x[0]
    w1 = W1.astype(jnp.bfloat16)
    w2 = W2.astype(jnp.bfloat16)

    slots2, wrows, meta2d = pl.pallas_call(
        functools.partial(_router_kernel, n_experts=E, n_blocks=NB, seq=S),
        in_specs=[pl.BlockSpec((S, H), lambda: (0, 0)),
                  pl.BlockSpec((E, H), lambda: (0, 0))],
        out_specs=[pl.BlockSpec((2, S), lambda: (0, 0)),
                   pl.BlockSpec((2 * S, 128), lambda: (0, 0)),
                   pl.BlockSpec((1, NB + 1), lambda: (0, 0))],
        out_shape=[jax.ShapeDtypeStruct((2, S), jnp.int32),
                   jax.ShapeDtypeStruct((2 * S, 128), jnp.float32),
                   jax.ShapeDtypeStruct((1, NB + 1), jnp.int32)],
    )(xs, Wg)

    meta = meta2d.reshape(NB + 1)

    mesh = plsc.VectorSubcoreMesh(core_axis_name="c", subcore_axis_name="s")
    NW = 32
    tok_per = S // NW

    @functools.partial(
        pl.kernel, mesh=mesh,
        out_type=[jax.ShapeDtypeStruct((G, H), jnp.float32),
                  jax.ShapeDtypeStruct((G, 128), jnp.float32)],
        scratch_types=[pltpu.VMEM((tok_per,), jnp.int32),
                       pltpu.VMEM((tok_per,), jnp.int32),
                       pltpu.VMEM((tok_per, H), jnp.float32),
                       pltpu.VMEM((tok_per, 128), jnp.float32),
                       pltpu.VMEM((tok_per, 128), jnp.float32),
                       pltpu.SemaphoreType.DMA,
                       pltpu.SemaphoreType.DMA])
    def _dispatch(x_hbm, s_hbm, w_hbm, xg_hbm, wv_hbm,
                  idx0_v, idx1_v, rows_v, w0_v, w1_v, lsem, ssem):
        wid = lax.axis_index("s") * 2 + lax.axis_index("c")
        base = wid * tok_per
        l0 = pltpu.async_copy(s_hbm.at[0, pl.ds(base, tok_per)], idx0_v, lsem)
        l1 = pltpu.async_copy(s_hbm.at[1, pl.ds(base, tok_per)], idx1_v, lsem)
        l2 = pltpu.async_copy(x_hbm.at[pl.ds(base, tok_per)], rows_v, lsem)
        l3 = pltpu.async_copy(w_hbm.at[pl.ds(base, tok_per)], w0_v, lsem)
        l4 = pltpu.async_copy(w_hbm.at[pl.ds(S + base, tok_per)], w1_v, lsem)
        l0.wait(); l1.wait(); l2.wait(); l3.wait(); l4.wait()
        s0 = pltpu.async_copy(rows_v, xg_hbm.at[idx0_v], ssem)
        s1 = pltpu.async_copy(rows_v, xg_hbm.at[idx1_v], ssem)
        s2 = pltpu.async_copy(w0_v, wv_hbm.at[idx0_v], ssem)
        s3 = pltpu.async_copy(w1_v, wv_hbm.at[idx1_v], ssem)
        s0.wait(); s1.wait(); s2.wait(); s3.wait()

    xg, wvec = _dispatch(xs, slots2, wrows)

    ygw = pl.pallas_call(
        functools.partial(_gemm_kernel, n_blocks=NB, seq=S),
        grid_spec=pltpu.PrefetchScalarGridSpec(
            num_scalar_prefetch=1,
            grid=(NB,),
            in_specs=[
                pl.BlockSpec((BLK, 128), lambda i, meta: (i, 0)),
                pl.BlockSpec((BLK, H), lambda i, meta: (i, 0)),
                pl.BlockSpec((1, F, H), lambda i, meta: (meta[i], 0, 0)),
                pl.BlockSpec((1, 1, F), lambda i, meta: (meta[i], 0, 0)),
                pl.BlockSpec((1, H, F), lambda i, meta: (meta[i], 0, 0)),
                pl.BlockSpec((1, 1, H), lambda i, meta: (meta[i], 0, 0)),
            ],
            out_specs=pl.BlockSpec((BLK, H), lambda i, meta: (i, 0)),
        ),
        compiler_params=pltpu.CompilerParams(
            dimension_semantics=("parallel",)),
        out_shape=jax.ShapeDtypeStruct((G, H), jnp.float32),
    )(meta, wvec, xg, w1, b1.reshape(E, 1, F), w2,
      b2.reshape(E, 1, H))

    CH = 64

    @functools.partial(
        pl.kernel, mesh=mesh,
        out_type=jax.ShapeDtypeStruct((2 * S, H), jnp.float32),
        scratch_types=[pltpu.VMEM((CH,), jnp.int32),
                       pltpu.VMEM((CH, H), jnp.float32)])
    def _combine(ygw_hbm, s_hbm, y_hbm, idx_v, rows_v):
        wid = lax.axis_index("s") * 2 + lax.axis_index("c")
        k = wid % 2
        tchunk = wid // 2
        for c in range(2):
            base = tchunk * 2 * CH + c * CH
            pltpu.sync_copy(s_hbm.at[k, pl.ds(base, CH)], idx_v)
            pltpu.sync_copy(ygw_hbm.at[idx_v], rows_v)
            pltpu.sync_copy(rows_v, y_hbm.at[pl.ds(k * S + base, CH)])

    y01 = _combine(ygw, slots2)

    TS = 1024
    NT = S // TS
    out = pl.pallas_call(
        _add_kernel,
        grid=(NT,),
        in_specs=[pl.BlockSpec((TS, H), lambda t: (t, 0)),
                  pl.BlockSpec((TS, H), lambda t: (t + NT, 0))],
        out_specs=pl.BlockSpec((TS, H), lambda t: (t, 0)),
        compiler_params=pltpu.CompilerParams(
            dimension_semantics=("parallel",)),
        out_shape=jax.ShapeDtypeStruct((S, H), jnp.float32),
    )(y01, y01)
    return out[None]
